# Initial kernel scaffold; baseline (speedup 1.0000x reference)
import jax
import jax.numpy as jnp
from jax.experimental import pallas as pl


def kernel(x, edge_index, edge_attr, problemType, batch, Wl, Wr, att, bias_g, fc1_W, fc1_b, fc2_W, fc2_b, fc3_W, fc3_b):
    raise NotImplementedError("write your pallas kernel here")


# speedup vs baseline: 3.5666x; 3.5666x over previous
import functools

import jax
import jax.numpy as jnp
from jax import lax
from jax.experimental import pallas as pl
from jax.experimental.pallas import tpu as pltpu
from jax.experimental.pallas import tpu_sc as plsc

_N = 10000
_E = 320000
_D = 128
_S = 2
_NSUB = 16
_EW = _E // _NSUB
_C = 32
_NCH = _EW // _C
_RPS = 624
_TAIL0 = _RPS * _NSUB
_TAILN = _N - _TAIL0


def _leaky02(v):
    return jnp.where(v > 0.0, v, v * 0.2)


def _edge_body(att_hbm, xl_hbm, xr_hbm, src_hbm, dst_hbm, ea_hbm,
               out_hbm, w_hbm,
               den_sh, out_sh,
               lbuf, rbuf, srcc, dstc, eac, wc, accbuf, attbuf, denbuf,
               coefbuf, zbuf, sem1, sem2):
    c = lax.axis_index("c")
    sid = lax.axis_index("s")
    base = sid * _EW

    pltpu.sync_copy(att_hbm.at[c], attbuf)
    attk = [attbuf[pl.ds(k * 16, 16)] for k in range(8)]

    zv = jnp.zeros((16,), jnp.float32)
    for r in range(_C):
        for k in range(8):
            lbuf[r, pl.ds(k * 16, 16)] = zv
    for k in range(64):
        zbuf[pl.ds(k * 16, 16)] = zv
    r0 = sid * _RPS
    for t in range(_RPS // _C):
        pltpu.sync_copy(lbuf, out_sh.at[pl.ds(r0 + t * _C, _C)])
    rem = _RPS - (_RPS // _C) * _C
    pltpu.sync_copy(lbuf.at[pl.ds(0, rem)],
                    out_sh.at[pl.ds(r0 + (_RPS // _C) * _C, rem)])

    @pl.when(sid == _NSUB - 1)
    def _zero_tail():
        pltpu.sync_copy(lbuf.at[pl.ds(0, _TAILN)],
                        out_sh.at[pl.ds(_TAIL0, _TAILN)])

    @pl.when(sid < 10)
    def _zero_den():
        pltpu.sync_copy(zbuf.at[pl.ds(0, 1000)],
                        den_sh.at[pl.ds(sid * 1000, 1000)])

    plsc.subcore_barrier()

    iota16 = lax.iota(jnp.int32, 16)

    def pass_a(i, _):
        ioff = base + i * _C
        pltpu.sync_copy(src_hbm.at[pl.ds(ioff, _C)], srcc)
        pltpu.sync_copy(dst_hbm.at[pl.ds(ioff, _C)], dstc)
        pltpu.sync_copy(ea_hbm.at[pl.ds(ioff, _C)], eac)
        cl = pltpu.async_copy(xl_hbm.at[c].at[srcc], lbuf, sem1)
        cr = pltpu.async_copy(xr_hbm.at[c].at[dstc], rbuf, sem2)
        cl.wait()
        cr.wait()
        for g in range(_C // 16):
            for j in range(16):
                e = g * 16 + j
                acc = jnp.zeros((16,), jnp.float32)
                for k in range(8):
                    sv = lbuf[e, pl.ds(k * 16, 16)] + rbuf[e, pl.ds(k * 16, 16)]
                    acc = acc + _leaky02(sv) * attk[k]
                accbuf[pl.ds(j * 16, 16)] = acc
            tot = jnp.zeros((16,), jnp.float32)
            for k in range(16):
                tot = tot + plsc.load_gather(accbuf, [iota16 * 16 + k])
            eav = eac[pl.ds(g * 16, 16)]
            wv = jnp.where(eav == c, jnp.exp(tot), 0.0)
            wc[pl.ds(g * 16, 16)] = wv
        pltpu.sync_copy(wc, w_hbm.at[c].at[pl.ds(ioff, _C)])
        pltpu.sync_copy(wc, den_sh.at[dstc], add=True)
        return 0

    lax.fori_loop(0, _NCH, pass_a, 0)
    plsc.subcore_barrier()

    def pass_b(i, _):
        ioff = base + i * _C
        pltpu.sync_copy(src_hbm.at[pl.ds(ioff, _C)], srcc)
        pltpu.sync_copy(dst_hbm.at[pl.ds(ioff, _C)], dstc)
        pltpu.sync_copy(w_hbm.at[c].at[pl.ds(ioff, _C)], wc)
        pltpu.sync_copy(den_sh.at[dstc], denbuf)
        cl = pltpu.async_copy(xl_hbm.at[c].at[srcc], lbuf, sem1)
        for g in range(_C // 16):
            cv = wc[pl.ds(g * 16, 16)] / (denbuf[pl.ds(g * 16, 16)] + 1e-16)
            coefbuf[pl.ds(g * 16, 16)] = cv
        cl.wait()
        for e in range(_C):
            cb = plsc.load_gather(coefbuf, [jnp.full((16,), e, jnp.int32)])
            for k in range(8):
                lbuf[e, pl.ds(k * 16, 16)] = lbuf[e, pl.ds(k * 16, 16)] * cb
        pltpu.sync_copy(lbuf, out_sh.at[dstc], add=True)
        return 0

    lax.fori_loop(0, _NCH, pass_b, 0)
    plsc.subcore_barrier()

    pltpu.sync_copy(out_sh.at[pl.ds(r0, _RPS)], out_hbm.at[c].at[pl.ds(r0, _RPS)])

    @pl.when(sid == _NSUB - 1)
    def _copy_tail():
        pltpu.sync_copy(out_sh.at[pl.ds(_TAIL0, _TAILN)],
                        out_hbm.at[c].at[pl.ds(_TAIL0, _TAILN)])


def _make_edge_call():
    mesh = plsc.VectorSubcoreMesh(core_axis_name="c", subcore_axis_name="s")
    return pl.kernel(
        _edge_body,
        out_type=(
            jax.ShapeDtypeStruct((_S, _N, _D), jnp.float32),
            jax.ShapeDtypeStruct((_S, _E), jnp.float32),
        ),
        mesh=mesh,
        compiler_params=pltpu.CompilerParams(needs_layout_passes=False),
        scratch_types=[
            pltpu.VMEM_SHARED((_N,), jnp.float32),
            pltpu.VMEM_SHARED((_N, _D), jnp.float32),
            pltpu.VMEM((_C, _D), jnp.float32),
            pltpu.VMEM((_C, _D), jnp.float32),
            pltpu.VMEM((_C,), jnp.int32),
            pltpu.VMEM((_C,), jnp.int32),
            pltpu.VMEM((_C,), jnp.int32),
            pltpu.VMEM((_C,), jnp.float32),
            pltpu.VMEM((256,), jnp.float32),
            pltpu.VMEM((_D,), jnp.float32),
            pltpu.VMEM((_C,), jnp.float32),
            pltpu.VMEM((_C,), jnp.float32),
            pltpu.VMEM((1024,), jnp.float32),
            pltpu.SemaphoreType.DMA,
            pltpu.SemaphoreType.DMA,
        ],
    )


_BN = 400


def _mm4_body(x_ref, w_ref, o_ref):
    o_ref[0] = jnp.dot(x_ref[...], w_ref[0],
                       preferred_element_type=jnp.float32)


def _mm4(x, w4):
    return pl.pallas_call(
        _mm4_body,
        grid=(4, _N // _BN),
        in_specs=[
            pl.BlockSpec((_BN, _D), lambda i, j: (j, 0)),
            pl.BlockSpec((1, _D, _D), lambda i, j: (i, 0, 0)),
        ],
        out_specs=pl.BlockSpec((1, _BN, _D), lambda i, j: (i, j, 0)),
        out_shape=jax.ShapeDtypeStruct((4, _N, _D), jnp.float32),
    )(x, w4)


def _combine_body(o2_ref, b_ref, xj_ref, cur_ref, xjo_ref):
    a0 = o2_ref[0] + b_ref[0:1, :]
    a1 = o2_ref[1] + b_ref[1:2, :]
    l0 = jnp.where(a0 > 0.0, a0, a0 * 0.01)
    l1 = jnp.where(a1 > 0.0, a1, a1 * 0.01)
    cur = (l0 + l1) * 0.5
    cur_ref[...] = cur
    xjo_ref[...] = jnp.maximum(xj_ref[...], cur)


def _combine(out2, bias2, xj):
    return pl.pallas_call(
        _combine_body,
        grid=(_N // _BN,),
        in_specs=[
            pl.BlockSpec((_S, _BN, _D), lambda j: (0, j, 0)),
            pl.BlockSpec((_S, _D), lambda j: (0, 0)),
            pl.BlockSpec((_BN, _D), lambda j: (j, 0)),
        ],
        out_specs=[
            pl.BlockSpec((_BN, _D), lambda j: (j, 0)),
            pl.BlockSpec((_BN, _D), lambda j: (j, 0)),
        ],
        out_shape=[
            jax.ShapeDtypeStruct((_N, _D), jnp.float32),
            jax.ShapeDtypeStruct((_N, _D), jnp.float32),
        ],
    )(out2, bias2, xj)


def _head_body(xj_ref, pt_ref, w1_ref, b1_ref, w2_ref, b2_ref, w3_ref, b3_ref,
               o_ref, acc_ref):
    j = pl.program_id(0)

    @pl.when(j == 0)
    def _init():
        acc_ref[...] = jnp.zeros_like(acc_ref)

    acc_ref[...] += jnp.sum(xj_ref[...], axis=0, keepdims=True)

    @pl.when(j == pl.num_programs(0) - 1)
    def _fin():
        pooled = acc_ref[...] * (1.0 / _N)
        h = jnp.dot(pooled, w1_ref[pl.ds(0, _D), :],
                    preferred_element_type=jnp.float32)
        h = h + pt_ref[0, 0] * w1_ref[pl.ds(_D, 1), :] + b1_ref[...]
        h = jnp.where(h > 0.0, h, h * 0.01)
        h = jnp.dot(h, w2_ref[...], preferred_element_type=jnp.float32) + b2_ref[...]
        h = jnp.where(h > 0.0, h, h * 0.01)
        o_ref[...] = jnp.dot(h, w3_ref[...],
                             preferred_element_type=jnp.float32) + b3_ref[...]


def _head(xj, pt, w1, b1, w2, b2, w3, b3):
    hid = w1.shape[1]
    out = w3.shape[1]
    return pl.pallas_call(
        _head_body,
        grid=(_N // _BN,),
        in_specs=[
            pl.BlockSpec((_BN, _D), lambda j: (j, 0)),
            pl.BlockSpec((1, 1), lambda j: (0, 0)),
            pl.BlockSpec((_D + 1, hid), lambda j: (0, 0)),
            pl.BlockSpec((1, hid), lambda j: (0, 0)),
            pl.BlockSpec((hid, hid), lambda j: (0, 0)),
            pl.BlockSpec((1, hid), lambda j: (0, 0)),
            pl.BlockSpec((hid, out), lambda j: (0, 0)),
            pl.BlockSpec((1, out), lambda j: (0, 0)),
        ],
        out_specs=pl.BlockSpec((1, out), lambda j: (0, 0)),
        out_shape=jax.ShapeDtypeStruct((1, out), jnp.float32),
        scratch_shapes=[pltpu.VMEM((1, _D), jnp.float32)],
    )(xj, pt, w1, b1, w2, b2, w3, b3)


def kernel(x, edge_index, edge_attr, problemType, batch, Wl, Wr, att, bias_g,
           fc1_W, fc1_b, fc2_W, fc2_b, fc3_W, fc3_b):
    src = edge_index[0]
    dst = edge_index[1]
    ea = edge_attr[:, 0].astype(jnp.int32)
    edge_call = _make_edge_call()

    cur = x
    xj = x
    for p in range(2):
        w4 = jnp.stack([Wl[p, 0], Wl[p, 1], Wr[p, 0], Wr[p, 1]])
        y = _mm4(cur, w4)
        att2 = att[p, :, 0, :]
        out2, _ = edge_call(att2, y[0:2], y[2:4], src, dst, ea)
        cur, xj = _combine(out2, bias_g[p], xj)

    pt = problemType.reshape(1, 1)
    return _head(xj, pt, fc1_W, fc1_b.reshape(1, -1), fc2_W,
                 fc2_b.reshape(1, -1), fc3_W, fc3_b.reshape(1, -1))


# speedup vs baseline: 5.2416x; 1.4696x over previous
import functools

import jax
import jax.numpy as jnp
from jax import lax
from jax.experimental import pallas as pl
from jax.experimental.pallas import tpu as pltpu
from jax.experimental.pallas import tpu_sc as plsc

_N = 10000
_E = 320000
_D = 128
_S = 2
_NSUB = 16
_EW = _E // _NSUB
_C = 32
_NCH = _EW // _C
_RPS = 624
_TAIL0 = _RPS * _NSUB
_TAILN = _N - _TAIL0


def _leaky02(v):
    return jnp.where(v > 0.0, v, v * 0.2)


def _edge_body(att_hbm, xl_hbm, xr_hbm, src_hbm, dst_hbm, ea_hbm,
               out_hbm, w_hbm,
               den_sh, out_sh,
               lbuf, rbuf, srcc, dstc, eac, wc, accbuf, attbuf, denbuf,
               coefbuf, zbuf, sem1, sem2, sem3, sem4, sem5):
    c = lax.axis_index("c")
    sid = lax.axis_index("s")
    base = sid * _EW

    pltpu.sync_copy(att_hbm.at[c], attbuf)
    attk = [attbuf[pl.ds(k * 16, 16)] for k in range(8)]

    zv = jnp.zeros((16,), jnp.float32)
    for r in range(_C):
        for k in range(8):
            lbuf[r, pl.ds(k * 16, 16)] = zv
    for k in range(64):
        zbuf[pl.ds(k * 16, 16)] = zv
    r0 = sid * _RPS
    for t in range(_RPS // _C):
        pltpu.sync_copy(lbuf, out_sh.at[pl.ds(r0 + t * _C, _C)])
    rem = _RPS - (_RPS // _C) * _C
    pltpu.sync_copy(lbuf.at[pl.ds(0, rem)],
                    out_sh.at[pl.ds(r0 + (_RPS // _C) * _C, rem)])

    @pl.when(sid == _NSUB - 1)
    def _zero_tail():
        pltpu.sync_copy(lbuf.at[pl.ds(0, _TAILN)],
                        out_sh.at[pl.ds(_TAIL0, _TAILN)])

    @pl.when(sid < 10)
    def _zero_den():
        pltpu.sync_copy(zbuf.at[pl.ds(0, 1000)],
                        den_sh.at[pl.ds(sid * 1000, 1000)])

    plsc.subcore_barrier()

    iota16 = lax.iota(jnp.int32, 16)

    def pass_a(i, _):
        ioff = base + i * _C
        ca = pltpu.async_copy(src_hbm.at[pl.ds(ioff, _C)], srcc, sem3)
        cb = pltpu.async_copy(dst_hbm.at[pl.ds(ioff, _C)], dstc, sem4)
        cc = pltpu.async_copy(ea_hbm.at[pl.ds(ioff, _C)], eac, sem5)
        ca.wait()
        cl = pltpu.async_copy(xl_hbm.at[c].at[srcc], lbuf, sem1)
        cb.wait()
        cr = pltpu.async_copy(xr_hbm.at[c].at[dstc], rbuf, sem2)
        cc.wait()
        cl.wait()
        cr.wait()
        for g in range(_C // 16):
            for j in range(16):
                e = g * 16 + j
                acc = jnp.zeros((16,), jnp.float32)
                for k in range(8):
                    sv = lbuf[e, pl.ds(k * 16, 16)] + rbuf[e, pl.ds(k * 16, 16)]
                    acc = acc + _leaky02(sv) * attk[k]
                accbuf[pl.ds(j * 16, 16)] = acc
            tot = jnp.zeros((16,), jnp.float32)
            for k in range(16):
                tot = tot + plsc.load_gather(accbuf, [iota16 * 16 + k])
            eav = eac[pl.ds(g * 16, 16)]
            wv = jnp.where(eav == c, jnp.exp(tot), 0.0)
            wc[pl.ds(g * 16, 16)] = wv
        cw = pltpu.async_copy(wc, w_hbm.at[c].at[pl.ds(ioff, _C)], sem3)
        pltpu.sync_copy(wc, den_sh.at[dstc], add=True)
        cw.wait()
        return 0

    lax.fori_loop(0, _NCH, pass_a, 0)
    plsc.subcore_barrier()

    def pass_b(i, _):
        ioff = base + i * _C
        ca = pltpu.async_copy(src_hbm.at[pl.ds(ioff, _C)], srcc, sem3)
        cb = pltpu.async_copy(dst_hbm.at[pl.ds(ioff, _C)], dstc, sem4)
        cw = pltpu.async_copy(w_hbm.at[c].at[pl.ds(ioff, _C)], wc, sem5)
        ca.wait()
        cl = pltpu.async_copy(xl_hbm.at[c].at[srcc], lbuf, sem1)
        cb.wait()
        cd = pltpu.async_copy(den_sh.at[dstc], denbuf, sem2)
        cw.wait()
        cd.wait()
        for g in range(_C // 16):
            cv = wc[pl.ds(g * 16, 16)] / (denbuf[pl.ds(g * 16, 16)] + 1e-16)
            coefbuf[pl.ds(g * 16, 16)] = cv
        cl.wait()
        for e in range(_C):
            cb = plsc.load_gather(coefbuf, [jnp.full((16,), e, jnp.int32)])
            for k in range(8):
                lbuf[e, pl.ds(k * 16, 16)] = lbuf[e, pl.ds(k * 16, 16)] * cb
        pltpu.sync_copy(lbuf, out_sh.at[dstc], add=True)
        return 0

    lax.fori_loop(0, _NCH, pass_b, 0)
    plsc.subcore_barrier()

    pltpu.sync_copy(out_sh.at[pl.ds(r0, _RPS)], out_hbm.at[c].at[pl.ds(r0, _RPS)])

    @pl.when(sid == _NSUB - 1)
    def _copy_tail():
        pltpu.sync_copy(out_sh.at[pl.ds(_TAIL0, _TAILN)],
                        out_hbm.at[c].at[pl.ds(_TAIL0, _TAILN)])


def _make_edge_call():
    mesh = plsc.VectorSubcoreMesh(core_axis_name="c", subcore_axis_name="s")
    return pl.kernel(
        _edge_body,
        out_type=(
            jax.ShapeDtypeStruct((_S, _N, _D), jnp.float32),
            jax.ShapeDtypeStruct((_S, _E), jnp.float32),
        ),
        mesh=mesh,
        compiler_params=pltpu.CompilerParams(needs_layout_passes=False),
        scratch_types=[
            pltpu.VMEM_SHARED((_N,), jnp.float32),
            pltpu.VMEM_SHARED((_N, _D), jnp.float32),
            pltpu.VMEM((_C, _D), jnp.float32),
            pltpu.VMEM((_C, _D), jnp.float32),
            pltpu.VMEM((_C,), jnp.int32),
            pltpu.VMEM((_C,), jnp.int32),
            pltpu.VMEM((_C,), jnp.int32),
            pltpu.VMEM((_C,), jnp.float32),
            pltpu.VMEM((256,), jnp.float32),
            pltpu.VMEM((_D,), jnp.float32),
            pltpu.VMEM((_C,), jnp.float32),
            pltpu.VMEM((_C,), jnp.float32),
            pltpu.VMEM((1024,), jnp.float32),
            pltpu.SemaphoreType.DMA,
            pltpu.SemaphoreType.DMA,
            pltpu.SemaphoreType.DMA,
            pltpu.SemaphoreType.DMA,
            pltpu.SemaphoreType.DMA,
        ],
    )


_BN = 400


def _mm4_body(x_ref, w_ref, o_ref):
    o_ref[0] = jnp.dot(x_ref[...], w_ref[0],
                       preferred_element_type=jnp.float32)


def _mm4(x, w4):
    return pl.pallas_call(
        _mm4_body,
        grid=(4, _N // _BN),
        in_specs=[
            pl.BlockSpec((_BN, _D), lambda i, j: (j, 0)),
            pl.BlockSpec((1, _D, _D), lambda i, j: (i, 0, 0)),
        ],
        out_specs=pl.BlockSpec((1, _BN, _D), lambda i, j: (i, j, 0)),
        out_shape=jax.ShapeDtypeStruct((4, _N, _D), jnp.float32),
    )(x, w4)


def _combine_body(o2_ref, b_ref, xj_ref, cur_ref, xjo_ref):
    a0 = o2_ref[0] + b_ref[0:1, :]
    a1 = o2_ref[1] + b_ref[1:2, :]
    l0 = jnp.where(a0 > 0.0, a0, a0 * 0.01)
    l1 = jnp.where(a1 > 0.0, a1, a1 * 0.01)
    cur = (l0 + l1) * 0.5
    cur_ref[...] = cur
    xjo_ref[...] = jnp.maximum(xj_ref[...], cur)


def _combine(out2, bias2, xj):
    return pl.pallas_call(
        _combine_body,
        grid=(_N // _BN,),
        in_specs=[
            pl.BlockSpec((_S, _BN, _D), lambda j: (0, j, 0)),
            pl.BlockSpec((_S, _D), lambda j: (0, 0)),
            pl.BlockSpec((_BN, _D), lambda j: (j, 0)),
        ],
        out_specs=[
            pl.BlockSpec((_BN, _D), lambda j: (j, 0)),
            pl.BlockSpec((_BN, _D), lambda j: (j, 0)),
        ],
        out_shape=[
            jax.ShapeDtypeStruct((_N, _D), jnp.float32),
            jax.ShapeDtypeStruct((_N, _D), jnp.float32),
        ],
    )(out2, bias2, xj)


def _head_body(xj_ref, pt_ref, w1_ref, b1_ref, w2_ref, b2_ref, w3_ref, b3_ref,
               o_ref, acc_ref):
    j = pl.program_id(0)

    @pl.when(j == 0)
    def _init():
        acc_ref[...] = jnp.zeros_like(acc_ref)

    acc_ref[...] += jnp.sum(xj_ref[...], axis=0, keepdims=True)

    @pl.when(j == pl.num_programs(0) - 1)
    def _fin():
        pooled = acc_ref[...] * (1.0 / _N)
        h = jnp.dot(pooled, w1_ref[pl.ds(0, _D), :],
                    preferred_element_type=jnp.float32)
        h = h + pt_ref[0, 0] * w1_ref[pl.ds(_D, 1), :] + b1_ref[...]
        h = jnp.where(h > 0.0, h, h * 0.01)
        h = jnp.dot(h, w2_ref[...], preferred_element_type=jnp.float32) + b2_ref[...]
        h = jnp.where(h > 0.0, h, h * 0.01)
        o_ref[...] = jnp.dot(h, w3_ref[...],
                             preferred_element_type=jnp.float32) + b3_ref[...]


def _head(xj, pt, w1, b1, w2, b2, w3, b3):
    hid = w1.shape[1]
    out = w3.shape[1]
    return pl.pallas_call(
        _head_body,
        grid=(_N // _BN,),
        in_specs=[
            pl.BlockSpec((_BN, _D), lambda j: (j, 0)),
            pl.BlockSpec((1, 1), lambda j: (0, 0)),
            pl.BlockSpec((_D + 1, hid), lambda j: (0, 0)),
            pl.BlockSpec((1, hid), lambda j: (0, 0)),
            pl.BlockSpec((hid, hid), lambda j: (0, 0)),
            pl.BlockSpec((1, hid), lambda j: (0, 0)),
            pl.BlockSpec((hid, out), lambda j: (0, 0)),
            pl.BlockSpec((1, out), lambda j: (0, 0)),
        ],
        out_specs=pl.BlockSpec((1, out), lambda j: (0, 0)),
        out_shape=jax.ShapeDtypeStruct((1, out), jnp.float32),
        scratch_shapes=[pltpu.VMEM((1, _D), jnp.float32)],
    )(xj, pt, w1, b1, w2, b2, w3, b3)


def kernel(x, edge_index, edge_attr, problemType, batch, Wl, Wr, att, bias_g,
           fc1_W, fc1_b, fc2_W, fc2_b, fc3_W, fc3_b):
    src = edge_index[0]
    dst = edge_index[1]
    ea = edge_attr[:, 0].astype(jnp.int32)
    edge_call = _make_edge_call()

    cur = x
    xj = x
    for p in range(2):
        w4 = jnp.stack([Wl[p, 0], Wl[p, 1], Wr[p, 0], Wr[p, 1]])
        y = _mm4(cur, w4)
        att2 = att[p, :, 0, :]
        out2, _ = edge_call(att2, y[0:2], y[2:4], src, dst, ea)
        cur, xj = _combine(out2, bias_g[p], xj)

    pt = problemType.reshape(1, 1)
    return _head(xj, pt, fc1_W, fc1_b.reshape(1, -1), fc2_W,
                 fc2_b.reshape(1, -1), fc3_W, fc3_b.reshape(1, -1))
